# baseline (speedup 1.0000x reference)
import functools

import jax
import jax.numpy as jnp
from jax import lax
from jax.experimental import pallas as pl
from jax.experimental.pallas import tpu as pltpu
from jax.experimental.pallas import tpu_sc as plsc

B = 16384
D = 64
NC = 2
NS = 16
NW = NC * NS
BPW = B // NW
CHUNK = 256
NCHUNK = BPW // CHUNK


def _body(uid_h, iid_h, ue_h, imf_h, ie_h, vm_h, out_h,
          uidv, iidv, u_v, v_v, ie_v, m_v, out_v, sem):
    c = lax.axis_index("c")
    s = lax.axis_index("s")
    wid = s * NC + c
    base = wid * BPW

    pltpu.sync_copy(uid_h.at[pl.ds(base, BPW)], uidv)
    pltpu.sync_copy(iid_h.at[pl.ds(base, BPW)], iidv)

    for ck in range(NCHUNK):
        us = uidv.at[pl.ds(ck * CHUNK, CHUNK)]
        js = iidv.at[pl.ds(ck * CHUNK, CHUNK)]
        cp1 = pltpu.async_copy(ue_h.at[us], u_v, sem)
        cp2 = pltpu.async_copy(imf_h.at[js], v_v, sem)
        cp3 = pltpu.async_copy(ie_h.at[js], ie_v, sem)
        cp4 = pltpu.async_copy(vm_h.at[us], m_v, sem)
        cp1.wait()
        cp2.wait()
        cp3.wait()
        cp4.wait()

        lanes = lax.iota(jnp.int32, 16)

        def group(g, _):
            vec = jnp.zeros((16,), jnp.float32)
            for l in range(16):
                r = g * 16 + l
                acc = u_v[r, pl.ds(0, 16)] * v_v[r, pl.ds(0, 16)]
                acc = acc + ie_v[r, pl.ds(0, 16)] * m_v[r, pl.ds(0, 16)]
                for j in range(1, 4):
                    acc = acc + u_v[r, pl.ds(16 * j, 16)] * v_v[r, pl.ds(16 * j, 16)]
                    acc = acc + ie_v[r, pl.ds(16 * j, 16)] * m_v[r, pl.ds(16 * j, 16)]
                vec = jnp.where(lanes == l, jnp.sum(acc), vec)
            out_v[pl.ds(ck * CHUNK + g * 16, 16)] = vec
            return 0

        lax.fori_loop(0, CHUNK // 16, group, 0)

    pltpu.sync_copy(out_v, out_h.at[pl.ds(base, BPW)])


def kernel(uid, iid, user_emb, item_mf_emb, item_emb, vae_mean):
    mesh = plsc.VectorSubcoreMesh(core_axis_name="c", subcore_axis_name="s")
    k = functools.partial(
        pl.kernel,
        out_type=jax.ShapeDtypeStruct((B,), jnp.float32),
        mesh=mesh,
        compiler_params=pltpu.CompilerParams(
            needs_layout_passes=False, use_tc_tiling_on_sc=False),
        scratch_types=[
            pltpu.VMEM((BPW,), jnp.int32),
            pltpu.VMEM((BPW,), jnp.int32),
            pltpu.VMEM((CHUNK, D), jnp.float32),
            pltpu.VMEM((CHUNK, D), jnp.float32),
            pltpu.VMEM((CHUNK, D), jnp.float32),
            pltpu.VMEM((CHUNK, D), jnp.float32),
            pltpu.VMEM((BPW,), jnp.float32),
            pltpu.SemaphoreType.DMA,
        ],
    )(_body)
    return k(uid.astype(jnp.int32), iid.astype(jnp.int32),
             user_emb, item_mf_emb, item_emb, vae_mean)
